# baseline (speedup 1.0000x reference)
import functools
import math

import jax
import jax.numpy as jnp
from jax import lax
from jax.experimental import pallas as pl
from jax.experimental.pallas import tpu as pltpu
from jax.experimental.pallas import tpu_sc as plsc

_NC = 2
_NS = 16
_NW = _NC * _NS
_W = 3584
_PR = 4
_T = 4


def _build_sc_pass(n_pad, rows_per_tile, with_deg):
    ri = n_pad // _NS
    mesh = plsc.VectorSubcoreMesh(core_axis_name="c", subcore_axis_name="s")
    out_type = [jax.ShapeDtypeStruct((_NC, _T, n_pad), jnp.float32)]
    if with_deg:
        out_type.append(jax.ShapeDtypeStruct((_NC, n_pad), jnp.float32))

    nparts = rows_per_tile // _PR
    assert nparts * _PR == rows_per_tile

    def body(src_hbm, dst_hbm, x_hbm, zd_hbm, *refs):
        if with_deg:
            s_out, d_out = refs[0], refs[1]
            refs = refs[2:]
        else:
            s_out = refs[0]
            refs = refs[1:]
        x_sh = refs[0:_T]
        s_sh = refs[_T:2 * _T]
        d_sh, sidx, didx, ones, vbuf, sem_g, sem_s = refs[2 * _T:2 * _T + 7]
        rows = refs[2 * _T + 7:]
        c = lax.axis_index("c")
        s = lax.axis_index("s")
        wid = c * _NS + s

        r0 = s * ri
        for t in range(_T):
            pltpu.sync_copy(x_hbm.at[t, pl.ds(r0, ri)], vbuf)
            pltpu.sync_copy(vbuf, x_sh[t].at[pl.ds(r0, ri)])
        pltpu.sync_copy(zd_hbm.at[pl.ds(r0, ri)], vbuf)
        for t in range(_T):
            pltpu.sync_copy(vbuf, s_sh[t].at[pl.ds(r0, ri)])
        pltpu.sync_copy(vbuf, d_sh.at[pl.ds(r0, ri)])
        if with_deg:
            for i in range(_W // 16):
                ones[pl.ds(i * 16, 16)] = jnp.ones((16,), jnp.float32)
        plsc.subcore_barrier()

        row_base = wid * rows_per_tile

        zsrc = zd_hbm.at[pl.ds(0, _W)]
        n_sc = _T + (1 if with_deg else 0)

        def _drain(rset):
            for t in range(_T):
                pltpu.make_async_copy(zsrc, rows[rset * _T + t], sem_s).wait()
            if with_deg:
                pltpu.make_async_copy(zsrc, ones, sem_s).wait()

        def _half(m, rset):
            mo = m * _W
            _drain(rset)
            gs = [pltpu.async_copy(x_sh[t].at[sidx.at[pl.ds(mo, _W)]],
                                   rows[rset * _T + t], sem_g)
                  for t in range(_T)]
            return gs

        def _scatters(m, rset):
            mo = m * _W
            for t in range(_T):
                pltpu.async_copy(rows[rset * _T + t],
                                 s_sh[t].at[didx.at[pl.ds(mo, _W)]],
                                 sem_s, add=True)
            if with_deg:
                pltpu.async_copy(ones, d_sh.at[sidx.at[pl.ds(mo, _W)]],
                                 sem_s, add=True)

        def blk(i, carry):
            m0 = 2 * i
            g0 = _half(m0, 0)
            g1 = _half(m0 + 1, 1)
            for cp in g0:
                cp.wait()
            _scatters(m0, 0)
            for cp in g1:
                cp.wait()
            _scatters(m0 + 1, 1)
            return carry

        part_edges = _PR * _W
        zv = vbuf.at[pl.ds(0, _W)]
        for h in range(nparts):
            hb = (row_base + h * _PR) * _W
            pltpu.sync_copy(src_hbm.at[pl.ds(hb, part_edges)], sidx)
            pltpu.sync_copy(dst_hbm.at[pl.ds(hb, part_edges)], didx)
            for _ in range(2):
                for t in range(_T):
                    pltpu.async_copy(zv, s_sh[t].at[sidx.at[pl.ds(0, _W)]],
                                     sem_s, add=True)
                if with_deg:
                    pltpu.async_copy(zv, d_sh.at[sidx.at[pl.ds(0, _W)]],
                                     sem_s, add=True)
            lax.fori_loop(0, _PR // 2, blk, 0)
            for rset in range(2):
                _drain(rset)
        plsc.subcore_barrier()

        for t in range(_T):
            pltpu.sync_copy(s_sh[t].at[pl.ds(r0, ri)], vbuf)
            pltpu.sync_copy(vbuf, s_out.at[c, t, pl.ds(r0, ri)])
        if with_deg:
            pltpu.sync_copy(d_sh.at[pl.ds(r0, ri)], vbuf)
            pltpu.sync_copy(vbuf, d_out.at[c, pl.ds(r0, ri)])

    scratch = (
        [pltpu.VMEM_SHARED((n_pad,), jnp.float32) for _ in range(_T)]
        + [pltpu.VMEM_SHARED((n_pad,), jnp.float32) for _ in range(_T)]
        + [pltpu.VMEM_SHARED((n_pad,), jnp.float32),
           pltpu.VMEM((_PR * _W,), jnp.int32),
           pltpu.VMEM((_PR * _W,), jnp.int32),
           pltpu.VMEM((_W,), jnp.float32),
           pltpu.VMEM((n_pad // _NS,), jnp.float32),
           pltpu.SemaphoreType.DMA,
           pltpu.SemaphoreType.DMA]
        + [pltpu.VMEM((_W,), jnp.float32)
           for _ in range(2 * _T)]
    )
    return pl.kernel(body, out_type=tuple(out_type), mesh=mesh,
                     scratch_types=scratch)


def _combine(x_p, s_parts, deg, a1, g, b):
    p = jax.nn.sigmoid(g)[0, 0]
    sw = jnp.exp(a1)[0, 0]
    nw = sw * jnp.tanh(a1)[0, 0]
    ldeg = jnp.log(jnp.maximum(deg, 1.0))
    ws = jnp.exp(p * ldeg)[None, :]
    wn = jnp.exp((p - 1.0) * ldeg)[None, :]
    s_sum = s_parts[0] + s_parts[1]
    return sw * x_p * ws + nw * wn * s_sum + b[0, 0]


def kernel(x, edge_index, alpha1_0, alpha2_0, gamma_0, bias_0,
           alpha1_1, alpha2_1, gamma_1, bias_1):
    t, n = x.shape
    e = edge_index.shape[1]
    blk_edges = _NW * _PR * _W
    e_pad = math.ceil(e / blk_edges) * blk_edges
    ept = e_pad // _NW
    n_pad = ((n + 256 + 255) // 256) * 256

    pe = e_pad - e
    pad_idx = (jnp.arange(pe, dtype=jnp.int32) % 256) + n
    src2d = jnp.concatenate([edge_index[0], pad_idx])
    dst2d = jnp.concatenate([edge_index[1], pad_idx])

    x_p = jnp.concatenate([x, jnp.zeros((t, n_pad - n), jnp.float32)], axis=1)
    zd = jnp.zeros((n_pad,), jnp.float32)

    pass1 = _build_sc_pass(n_pad, ept // _W, with_deg=True)
    pass2 = _build_sc_pass(n_pad, ept // _W, with_deg=False)

    s1, degp = pass1(src2d, dst2d, x_p, zd)
    deg = degp[0] + degp[1]
    x2 = _combine(x_p, s1, deg, alpha1_0, gamma_0, bias_0)
    (s2,) = pass2(src2d, dst2d, x2, zd)
    out = _combine(x2, s2, deg, alpha1_1, gamma_1, bias_1)
    return out[:, :n]


# speedup vs baseline: 1.4271x; 1.4271x over previous
import functools
import math

import jax
import jax.numpy as jnp
from jax import lax
from jax.experimental import pallas as pl
from jax.experimental.pallas import tpu as pltpu
from jax.experimental.pallas import tpu_sc as plsc

_NC = 2
_NS = 16
_NW = _NC * _NS
_W = 7168
_PR = 2
_T = 4


def _build_sc_pass(n_pad, rows_per_tile, with_deg):
    ri = n_pad // _NS
    mesh = plsc.VectorSubcoreMesh(core_axis_name="c", subcore_axis_name="s")
    out_type = [jax.ShapeDtypeStruct((_NC, _T, n_pad), jnp.float32)]
    if with_deg:
        out_type.append(jax.ShapeDtypeStruct((_NC, n_pad), jnp.float32))

    nparts = rows_per_tile // _PR
    assert nparts * _PR == rows_per_tile

    def body(src_hbm, dst_hbm, x_hbm, zd_hbm, *refs):
        if with_deg:
            s_out, d_out = refs[0], refs[1]
            refs = refs[2:]
        else:
            s_out = refs[0]
            refs = refs[1:]
        x_sh = refs[0:_T]
        s_sh = refs[_T:2 * _T]
        d_sh, sidx, didx, ones, vbuf, sem_g, sem_s = refs[2 * _T:2 * _T + 7]
        rows = refs[2 * _T + 7:]
        c = lax.axis_index("c")
        s = lax.axis_index("s")
        wid = c * _NS + s

        r0 = s * ri
        for t in range(_T):
            pltpu.sync_copy(x_hbm.at[t, pl.ds(r0, ri)], vbuf)
            pltpu.sync_copy(vbuf, x_sh[t].at[pl.ds(r0, ri)])
        pltpu.sync_copy(zd_hbm.at[pl.ds(r0, ri)], vbuf)
        for t in range(_T):
            pltpu.sync_copy(vbuf, s_sh[t].at[pl.ds(r0, ri)])
        pltpu.sync_copy(vbuf, d_sh.at[pl.ds(r0, ri)])
        if with_deg:
            for i in range(_W // 16):
                ones[pl.ds(i * 16, 16)] = jnp.ones((16,), jnp.float32)
        plsc.subcore_barrier()

        row_base = wid * rows_per_tile

        def blk(m, carry):
            mo = m * _W
            gs = [pltpu.async_copy(x_sh[t].at[sidx.at[pl.ds(mo, _W)]],
                                   rows[t], sem_g)
                  for t in range(_T)]
            dgs = []
            if with_deg:
                dgs = [pltpu.async_copy(ones, d_sh.at[sidx.at[pl.ds(mo, _W)]],
                                        sem_s, add=True)]
            for cp in gs:
                cp.wait()
            ss = [pltpu.async_copy(rows[t],
                                   s_sh[t].at[didx.at[pl.ds(mo, _W)]],
                                   sem_s, add=True)
                  for t in range(_T)]
            for cp in dgs + ss:
                cp.wait()
            return carry

        part_edges = _PR * _W
        for h in range(nparts):
            hb = (row_base + h * _PR) * _W
            pltpu.sync_copy(src_hbm.at[pl.ds(hb, part_edges)], sidx)
            pltpu.sync_copy(dst_hbm.at[pl.ds(hb, part_edges)], didx)
            lax.fori_loop(0, _PR, blk, 0)
        plsc.subcore_barrier()

        for t in range(_T):
            pltpu.sync_copy(s_sh[t].at[pl.ds(r0, ri)], vbuf)
            pltpu.sync_copy(vbuf, s_out.at[c, t, pl.ds(r0, ri)])
        if with_deg:
            pltpu.sync_copy(d_sh.at[pl.ds(r0, ri)], vbuf)
            pltpu.sync_copy(vbuf, d_out.at[c, pl.ds(r0, ri)])

    scratch = (
        [pltpu.VMEM_SHARED((n_pad,), jnp.float32) for _ in range(_T)]
        + [pltpu.VMEM_SHARED((n_pad,), jnp.float32) for _ in range(_T)]
        + [pltpu.VMEM_SHARED((n_pad,), jnp.float32),
           pltpu.VMEM((_PR * _W,), jnp.int32),
           pltpu.VMEM((_PR * _W,), jnp.int32),
           pltpu.VMEM((_W,), jnp.float32),
           pltpu.VMEM((n_pad // _NS,), jnp.float32),
           pltpu.SemaphoreType.DMA,
           pltpu.SemaphoreType.DMA]
        + [pltpu.VMEM((_W,), jnp.float32)
           for _ in range(_T)]
    )
    return pl.kernel(body, out_type=tuple(out_type), mesh=mesh,
                     scratch_types=scratch)


def _combine(x_p, s_parts, deg, a1, g, b):
    p = jax.nn.sigmoid(g)[0, 0]
    sw = jnp.exp(a1)[0, 0]
    nw = sw * jnp.tanh(a1)[0, 0]
    ldeg = jnp.log(jnp.maximum(deg, 1.0))
    ws = jnp.exp(p * ldeg)[None, :]
    wn = jnp.exp((p - 1.0) * ldeg)[None, :]
    s_sum = s_parts[0] + s_parts[1]
    return sw * x_p * ws + nw * wn * s_sum + b[0, 0]


def kernel(x, edge_index, alpha1_0, alpha2_0, gamma_0, bias_0,
           alpha1_1, alpha2_1, gamma_1, bias_1):
    t, n = x.shape
    e = edge_index.shape[1]
    blk_edges = _NW * _PR * _W
    e_pad = math.ceil(e / blk_edges) * blk_edges
    ept = e_pad // _NW
    n_pad = ((n + 256 + 255) // 256) * 256

    pe = e_pad - e
    pad_idx = (jnp.arange(pe, dtype=jnp.int32) % 256) + n
    src2d = jnp.concatenate([edge_index[0], pad_idx])
    dst2d = jnp.concatenate([edge_index[1], pad_idx])

    x_p = jnp.concatenate([x, jnp.zeros((t, n_pad - n), jnp.float32)], axis=1)
    zd = jnp.zeros((n_pad,), jnp.float32)

    pass1 = _build_sc_pass(n_pad, ept // _W, with_deg=True)
    pass2 = _build_sc_pass(n_pad, ept // _W, with_deg=False)

    s1, degp = pass1(src2d, dst2d, x_p, zd)
    deg = degp[0] + degp[1]
    x2 = _combine(x_p, s1, deg, alpha1_0, gamma_0, bias_0)
    (s2,) = pass2(src2d, dst2d, x2, zd)
    out = _combine(x2, s2, deg, alpha1_1, gamma_1, bias_1)
    return out[:, :n]
